# baseline (speedup 1.0000x reference)
import functools

import jax
import jax.numpy as jnp
from jax.experimental import pallas as pl
from jax.experimental.pallas import tpu as pltpu

DIM = 2048
INTER = 1024
E = 64

TM = 64
TR = 256


def _router_body(x_ref, gw_ref, eid_ref, g_ref):
    logits = jax.lax.dot_general(
        gw_ref[...], x_ref[...],
        dimension_numbers=(((1,), (1,)), ((), ())),
        preferred_element_type=jnp.float32)
    scores = jax.nn.sigmoid(logits)
    eid = jnp.argmax(scores, axis=0).astype(jnp.int32)
    smax = jnp.max(scores, axis=0)
    g = smax / jnp.maximum(smax, 1e-12)
    eid_ref[0, 0, :] = eid
    g_ref[0, 0, :] = g


def _route(xf, gate_w, t):
    nr = t // TR
    eid3, g3 = pl.pallas_call(
        _router_body,
        grid=(nr,),
        in_specs=[
            pl.BlockSpec((TR, DIM), lambda r: (r, 0)),
            pl.BlockSpec((E, DIM), lambda r: (0, 0)),
        ],
        out_specs=[
            pl.BlockSpec((1, 1, TR), lambda r: (r, 0, 0)),
            pl.BlockSpec((1, 1, TR), lambda r: (r, 0, 0)),
        ],
        out_shape=[
            jax.ShapeDtypeStruct((nr, 1, TR), jnp.int32),
            jax.ShapeDtypeStruct((nr, 1, TR), jnp.float32),
        ],
    )(xf, gate_w)
    return eid3.reshape(t), g3.reshape(t)


def _gmm_body(sm_ref, se_ref, sfl_ref, soff_ref,
              x_ref, w1_ref, w3_ref, w2_ref, sw1_ref, sw3_ref, sw2_ref,
              g_ref, out_ref):
    s = pl.program_id(0)
    e = se_ref[s]
    fl = sfl_ref[s]
    start = soff_ref[e]
    end = soff_ref[e + 1]
    row0 = sm_ref[s] * TM

    x = x_ref[...]

    @pl.when((fl & 2) != 0)
    def _shared():
        h1 = jax.lax.dot_general(
            x, sw1_ref[...], (((1,), (1,)), ((), ())),
            preferred_element_type=jnp.float32)
        h3 = jax.lax.dot_general(
            x, sw3_ref[...], (((1,), (1,)), ((), ())),
            preferred_element_type=jnp.float32)
        hh = (h1 * jax.nn.sigmoid(h1) * h3).astype(jnp.bfloat16)
        o = jax.lax.dot_general(
            hh, sw2_ref[...], (((1,), (1,)), ((), ())),
            preferred_element_type=jnp.float32)
        out_ref[...] = o.astype(jnp.bfloat16)

    @pl.when((fl & 1) != 0)
    def _expert():
        h1 = jnp.dot(x, w1_ref[0], preferred_element_type=jnp.float32)
        h3 = jnp.dot(x, w3_ref[0], preferred_element_type=jnp.float32)
        hh = (h1 * jax.nn.sigmoid(h1) * h3).astype(jnp.bfloat16)
        o = jnp.dot(hh, w2_ref[0], preferred_element_type=jnp.float32)
        rows = row0 + jax.lax.broadcasted_iota(jnp.int32, (TM, 1), 0)
        mask = (rows >= start) & (rows < end)
        gcol = g_ref[0, 0, :].reshape(TM, 1)
        contrib = jnp.where(mask, o * gcol, 0.0)
        out_ref[...] = (out_ref[...].astype(jnp.float32)
                        + contrib).astype(jnp.bfloat16)


def _gmm(xs, w1, w3, w2, sw1, sw3, sw2, g3, sm, se, sfl, soff, t):
    nsteps = sm.shape[0]
    grid_spec = pltpu.PrefetchScalarGridSpec(
        num_scalar_prefetch=4,
        grid=(nsteps,),
        in_specs=[
            pl.BlockSpec((TM, DIM), lambda s, sm, se, sfl, soff: (sm[s], 0)),
            pl.BlockSpec((1, DIM, INTER),
                         lambda s, sm, se, sfl, soff: (se[s], 0, 0)),
            pl.BlockSpec((1, DIM, INTER),
                         lambda s, sm, se, sfl, soff: (se[s], 0, 0)),
            pl.BlockSpec((1, INTER, DIM),
                         lambda s, sm, se, sfl, soff: (se[s], 0, 0)),
            pl.BlockSpec((INTER, DIM), lambda s, sm, se, sfl, soff: (0, 0)),
            pl.BlockSpec((INTER, DIM), lambda s, sm, se, sfl, soff: (0, 0)),
            pl.BlockSpec((DIM, INTER), lambda s, sm, se, sfl, soff: (0, 0)),
            pl.BlockSpec((1, 1, TM),
                         lambda s, sm, se, sfl, soff: (sm[s], 0, 0)),
        ],
        out_specs=pl.BlockSpec((TM, DIM),
                               lambda s, sm, se, sfl, soff: (sm[s], 0)),
    )
    return pl.pallas_call(
        _gmm_body,
        grid_spec=grid_spec,
        out_shape=jax.ShapeDtypeStruct((t, DIM), jnp.bfloat16),
    )(sm, se, sfl, soff, xs, w1, w3, w2, sw1, sw3, sw2, g3)


def _schedule(eid, t):
    ntiles = t // TM
    nsteps = ntiles + E - 1

    counts = jnp.bincount(eid, length=E).astype(jnp.int32)
    ends = jnp.cumsum(counts)
    starts = ends - counts
    soff = jnp.concatenate([jnp.zeros((1,), jnp.int32), ends]).astype(jnp.int32)

    perm = jnp.argsort(eid, stable=True)
    pos = jnp.zeros((t,), jnp.int32).at[perm].set(
        jnp.arange(t, dtype=jnp.int32))

    nonzero = counts > 0
    t_first = starts // TM
    t_last = jnp.where(nonzero, (ends - 1) // TM, 0)
    ntile_e = jnp.where(nonzero, t_last - t_first + 1, 0)
    cume = jnp.cumsum(ntile_e)
    cume_ex = cume - ntile_e
    total = cume[-1]

    sidx = jnp.arange(nsteps, dtype=jnp.int32)
    e_arr = jnp.clip(jnp.searchsorted(cume, sidx, side='right'),
                     0, E - 1).astype(jnp.int32)
    m_arr = (t_first[e_arr] + sidx - cume_ex[e_arr]).astype(jnp.int32)
    active = sidx < total

    e_last = jnp.max(jnp.where(nonzero, jnp.arange(E, dtype=jnp.int32), -1))
    m_arr = jnp.where(active, m_arr, ntiles - 1)
    e_arr = jnp.where(active, e_arr, e_last)

    first = jnp.concatenate(
        [jnp.ones((1,), jnp.bool_), m_arr[1:] != m_arr[:-1]])
    sfl = (active.astype(jnp.int32) + 2 * first.astype(jnp.int32))
    return perm, pos, soff, m_arr, e_arr, sfl


@jax.jit
def kernel(x, gate_w, W1, W3, W2, sw1, sw3, sw2):
    xf = x.reshape(-1, DIM)
    t = xf.shape[0]

    eid, g = _route(xf, gate_w, t)
    perm, pos, soff, sm, se, sfl = _schedule(eid, t)

    xs = jnp.take(xf, perm, axis=0)
    gs = jnp.take(g, perm).astype(jnp.bfloat16).astype(jnp.float32)
    g3 = gs.reshape(t // TM, 1, TM)

    out_sorted = _gmm(xs, W1, W3, W2, sw1, sw3, sw2, g3,
                      sm, se, sfl, soff, t)
    out = jnp.take(out_sorted, pos, axis=0)
    return out.reshape(x.shape)


# speedup vs baseline: 1.2040x; 1.2040x over previous
import functools

import jax
import jax.numpy as jnp
from jax.experimental import pallas as pl
from jax.experimental.pallas import tpu as pltpu

DIM = 2048
INTER = 1024
E = 64

TM = 256
TR = 256


def _router_body(x_ref, gw_ref, eid_ref, g_ref):
    logits = jax.lax.dot_general(
        gw_ref[...], x_ref[...],
        dimension_numbers=(((1,), (1,)), ((), ())),
        preferred_element_type=jnp.float32)
    scores = jax.nn.sigmoid(logits)
    eid = jnp.argmax(scores, axis=0).astype(jnp.int32)
    smax = jnp.max(scores, axis=0)
    g = smax / jnp.maximum(smax, 1e-12)
    eid_ref[0, 0, :] = eid
    g_ref[0, 0, :] = g


def _route(xf, gate_w, t):
    nr = t // TR
    eid3, g3 = pl.pallas_call(
        _router_body,
        grid=(nr,),
        in_specs=[
            pl.BlockSpec((TR, DIM), lambda r: (r, 0)),
            pl.BlockSpec((E, DIM), lambda r: (0, 0)),
        ],
        out_specs=[
            pl.BlockSpec((1, 1, TR), lambda r: (r, 0, 0)),
            pl.BlockSpec((1, 1, TR), lambda r: (r, 0, 0)),
        ],
        out_shape=[
            jax.ShapeDtypeStruct((nr, 1, TR), jnp.int32),
            jax.ShapeDtypeStruct((nr, 1, TR), jnp.float32),
        ],
    )(xf, gate_w)
    return eid3.reshape(t), g3.reshape(t)


def _gmm_body(sm_ref, se_ref, sfl_ref, soff_ref,
              x_ref, w1_ref, w3_ref, w2_ref, sw1_ref, sw3_ref, sw2_ref,
              g_ref, out_ref):
    s = pl.program_id(0)
    e = se_ref[s]
    fl = sfl_ref[s]
    start = soff_ref[e]
    end = soff_ref[e + 1]
    row0 = sm_ref[s] * TM

    x = x_ref[...]

    @pl.when((fl & 2) != 0)
    def _shared():
        h1 = jax.lax.dot_general(
            x, sw1_ref[...], (((1,), (1,)), ((), ())),
            preferred_element_type=jnp.float32)
        h3 = jax.lax.dot_general(
            x, sw3_ref[...], (((1,), (1,)), ((), ())),
            preferred_element_type=jnp.float32)
        hh = (h1 * jax.nn.sigmoid(h1) * h3).astype(jnp.bfloat16)
        o = jax.lax.dot_general(
            hh, sw2_ref[...], (((1,), (1,)), ((), ())),
            preferred_element_type=jnp.float32)
        out_ref[...] = o.astype(jnp.bfloat16)

    @pl.when((fl & 1) != 0)
    def _expert():
        h1 = jnp.dot(x, w1_ref[0], preferred_element_type=jnp.float32)
        h3 = jnp.dot(x, w3_ref[0], preferred_element_type=jnp.float32)
        hh = (h1 * jax.nn.sigmoid(h1) * h3).astype(jnp.bfloat16)
        o = jnp.dot(hh, w2_ref[0], preferred_element_type=jnp.float32)
        rows = row0 + jax.lax.broadcasted_iota(jnp.int32, (TM, 1), 0)
        mask = (rows >= start) & (rows < end)
        gcol = g_ref[0, 0, :].reshape(TM, 1)
        contrib = jnp.where(mask, o * gcol, 0.0)
        out_ref[...] = (out_ref[...].astype(jnp.float32)
                        + contrib).astype(jnp.bfloat16)


def _gmm(xs, w1, w3, w2, sw1, sw3, sw2, g3, sm, se, sfl, soff, t):
    nsteps = sm.shape[0]
    grid_spec = pltpu.PrefetchScalarGridSpec(
        num_scalar_prefetch=4,
        grid=(nsteps,),
        in_specs=[
            pl.BlockSpec((TM, DIM), lambda s, sm, se, sfl, soff: (sm[s], 0)),
            pl.BlockSpec((1, DIM, INTER),
                         lambda s, sm, se, sfl, soff: (se[s], 0, 0)),
            pl.BlockSpec((1, DIM, INTER),
                         lambda s, sm, se, sfl, soff: (se[s], 0, 0)),
            pl.BlockSpec((1, INTER, DIM),
                         lambda s, sm, se, sfl, soff: (se[s], 0, 0)),
            pl.BlockSpec((INTER, DIM), lambda s, sm, se, sfl, soff: (0, 0)),
            pl.BlockSpec((INTER, DIM), lambda s, sm, se, sfl, soff: (0, 0)),
            pl.BlockSpec((DIM, INTER), lambda s, sm, se, sfl, soff: (0, 0)),
            pl.BlockSpec((1, 1, TM),
                         lambda s, sm, se, sfl, soff: (sm[s], 0, 0)),
        ],
        out_specs=pl.BlockSpec((TM, DIM),
                               lambda s, sm, se, sfl, soff: (sm[s], 0)),
    )
    return pl.pallas_call(
        _gmm_body,
        grid_spec=grid_spec,
        out_shape=jax.ShapeDtypeStruct((t, DIM), jnp.bfloat16),
    )(sm, se, sfl, soff, xs, w1, w3, w2, sw1, sw3, sw2, g3)


def _schedule(eid, t):
    ntiles = t // TM
    nsteps = ntiles + E - 1

    counts = jnp.bincount(eid, length=E).astype(jnp.int32)
    ends = jnp.cumsum(counts)
    starts = ends - counts
    soff = jnp.concatenate([jnp.zeros((1,), jnp.int32), ends]).astype(jnp.int32)

    perm = jnp.argsort(eid, stable=True)
    pos = jnp.zeros((t,), jnp.int32).at[perm].set(
        jnp.arange(t, dtype=jnp.int32))

    nonzero = counts > 0
    t_first = starts // TM
    t_last = jnp.where(nonzero, (ends - 1) // TM, 0)
    ntile_e = jnp.where(nonzero, t_last - t_first + 1, 0)
    cume = jnp.cumsum(ntile_e)
    cume_ex = cume - ntile_e
    total = cume[-1]

    sidx = jnp.arange(nsteps, dtype=jnp.int32)
    e_arr = jnp.clip(jnp.searchsorted(cume, sidx, side='right'),
                     0, E - 1).astype(jnp.int32)
    m_arr = (t_first[e_arr] + sidx - cume_ex[e_arr]).astype(jnp.int32)
    active = sidx < total

    e_last = jnp.max(jnp.where(nonzero, jnp.arange(E, dtype=jnp.int32), -1))
    m_arr = jnp.where(active, m_arr, ntiles - 1)
    e_arr = jnp.where(active, e_arr, e_last)

    first = jnp.concatenate(
        [jnp.ones((1,), jnp.bool_), m_arr[1:] != m_arr[:-1]])
    sfl = (active.astype(jnp.int32) + 2 * first.astype(jnp.int32))
    return perm, pos, soff, m_arr, e_arr, sfl


@jax.jit
def kernel(x, gate_w, W1, W3, W2, sw1, sw3, sw2):
    xf = x.reshape(-1, DIM)
    t = xf.shape[0]

    eid, g = _route(xf, gate_w, t)
    perm, pos, soff, sm, se, sfl = _schedule(eid, t)

    xs = jnp.take(xf, perm, axis=0)
    gs = jnp.take(g, perm).astype(jnp.bfloat16).astype(jnp.float32)
    g3 = gs.reshape(t // TM, 1, TM)

    out_sorted = _gmm(xs, W1, W3, W2, sw1, sw3, sw2, g3,
                      sm, se, sfl, soff, t)
    out = jnp.take(out_sorted, pos, axis=0)
    return out.reshape(x.shape)


# speedup vs baseline: 1.2507x; 1.0387x over previous
import functools

import jax
import jax.numpy as jnp
from jax import lax
from jax.experimental import pallas as pl
from jax.experimental.pallas import tpu as pltpu
from jax.experimental.pallas import tpu_sc as plsc

DIM = 2048
INTER = 1024
E = 64

TM = 256
TR = 256


def _router_body(x_ref, gw_ref, eid_ref, g_ref):
    logits = jax.lax.dot_general(
        gw_ref[...], x_ref[...],
        dimension_numbers=(((1,), (1,)), ((), ())),
        preferred_element_type=jnp.float32)
    scores = jax.nn.sigmoid(logits)
    eid = jnp.argmax(scores, axis=0).astype(jnp.int32)
    smax = jnp.max(scores, axis=0)
    g = smax / jnp.maximum(smax, 1e-12)
    eid_ref[0, 0, :] = eid
    g_ref[0, 0, :] = g


def _route(xf, gate_w, t):
    nr = t // TR
    eid3, g3 = pl.pallas_call(
        _router_body,
        grid=(nr,),
        in_specs=[
            pl.BlockSpec((TR, DIM), lambda r: (r, 0)),
            pl.BlockSpec((E, DIM), lambda r: (0, 0)),
        ],
        out_specs=[
            pl.BlockSpec((1, 1, TR), lambda r: (r, 0, 0)),
            pl.BlockSpec((1, 1, TR), lambda r: (r, 0, 0)),
        ],
        out_shape=[
            jax.ShapeDtypeStruct((nr, 1, TR), jnp.int32),
            jax.ShapeDtypeStruct((nr, 1, TR), jnp.float32),
        ],
    )(xf, gate_w)
    return eid3.reshape(t), g3.reshape(t)


def _gmm_body(sm_ref, se_ref, sfl_ref, sstart_ref, send_ref,
              x_ref, w1_ref, w3_ref, w2_ref, sw1_ref, sw3_ref, sw2_ref,
              g_ref, out_ref):
    s = pl.program_id(0)
    fl = sfl_ref[s]
    start = sstart_ref[s]
    end = send_ref[s]
    row0 = sm_ref[s] * TM

    x = x_ref[...]

    @pl.when((fl & 2) != 0)
    def _shared():
        h1 = jax.lax.dot_general(
            x, sw1_ref[...], (((1,), (1,)), ((), ())),
            preferred_element_type=jnp.float32)
        h3 = jax.lax.dot_general(
            x, sw3_ref[...], (((1,), (1,)), ((), ())),
            preferred_element_type=jnp.float32)
        hh = (h1 * jax.nn.sigmoid(h1) * h3).astype(jnp.bfloat16)
        o = jax.lax.dot_general(
            hh, sw2_ref[...], (((1,), (1,)), ((), ())),
            preferred_element_type=jnp.float32)
        out_ref[...] = o.astype(jnp.bfloat16)

    @pl.when((fl & 1) != 0)
    def _expert():
        h1 = jnp.dot(x, w1_ref[0], preferred_element_type=jnp.float32)
        h3 = jnp.dot(x, w3_ref[0], preferred_element_type=jnp.float32)
        hh = (h1 * jax.nn.sigmoid(h1) * h3).astype(jnp.bfloat16)
        o = jnp.dot(hh, w2_ref[0], preferred_element_type=jnp.float32)
        rows = row0 + jax.lax.broadcasted_iota(jnp.int32, (TM, 1), 0)
        mask = (rows >= start) & (rows < end)
        gcol = g_ref[0, 0, :].reshape(TM, 1)
        contrib = jnp.where(mask, o * gcol, 0.0)
        out_ref[...] = (out_ref[...].astype(jnp.float32)
                        + contrib).astype(jnp.bfloat16)


def _gmm(xs, w1, w3, w2, sw1, sw3, sw2, g3, sm, se, sfl, sstart, send, t):
    nsteps = sm.shape[0]
    grid_spec = pltpu.PrefetchScalarGridSpec(
        num_scalar_prefetch=5,
        grid=(nsteps,),
        in_specs=[
            pl.BlockSpec((TM, DIM), lambda s, sm, se, *_: (sm[s], 0)),
            pl.BlockSpec((1, DIM, INTER), lambda s, sm, se, *_: (se[s], 0, 0)),
            pl.BlockSpec((1, DIM, INTER), lambda s, sm, se, *_: (se[s], 0, 0)),
            pl.BlockSpec((1, INTER, DIM), lambda s, sm, se, *_: (se[s], 0, 0)),
            pl.BlockSpec((INTER, DIM), lambda s, sm, se, *_: (0, 0)),
            pl.BlockSpec((INTER, DIM), lambda s, sm, se, *_: (0, 0)),
            pl.BlockSpec((DIM, INTER), lambda s, sm, se, *_: (0, 0)),
            pl.BlockSpec((1, 1, TM), lambda s, sm, se, *_: (sm[s], 0, 0)),
        ],
        out_specs=pl.BlockSpec((TM, DIM), lambda s, sm, se, *_: (sm[s], 0)),
    )
    return pl.pallas_call(
        _gmm_body,
        grid_spec=grid_spec,
        out_shape=jax.ShapeDtypeStruct((t, DIM), jnp.bfloat16),
    )(sm, se, sfl, sstart, send, xs, w1, w3, w2, sw1, sw3, sw2, g3)


def _dispatch(eid, t):
    info = plsc.get_sparse_core_info()
    nw = info.num_subcores
    ch = t // nw
    nbin = E // 16
    mesh = plsc.VectorSubcoreMesh(core_axis_name="c", subcore_axis_name="s",
                                  num_cores=1)

    @functools.partial(
        pl.kernel,
        out_type=[
            jax.ShapeDtypeStruct((t,), jnp.int32),
            jax.ShapeDtypeStruct((128,), jnp.int32),
            jax.ShapeDtypeStruct((nw, E), jnp.int32),
        ],
        mesh=mesh,
        scratch_types=[
            pltpu.VMEM((ch,), jnp.int32),
            pltpu.VMEM((E,), jnp.int32),
            pltpu.VMEM((nw, E), jnp.int32),
            pltpu.VMEM((ch,), jnp.int32),
            pltpu.VMEM((128,), jnp.int32),
        ],
    )
    def dispatch(eid_hbm, pos_hbm, soff_hbm, cnt_hbm,
                 eid_v, counts_v, all_counts_v, pos_v, soff_v):
        wid = lax.axis_index("s")

        pltpu.sync_copy(eid_hbm.at[pl.ds(wid * ch, ch)], eid_v)

        lane = lax.iota(jnp.int32, 16)

        def _gather16(v, idx):
            return v.at[idx].get(mode="promise_in_bounds")

        ev = [eid_v[pl.ds(16 * k, 16)] for k in range(ch // 16)]

        base = [jnp.zeros((16,), jnp.int32) for _ in range(nbin)]
        rank_vecs = []
        for k in range(ch // 16):
            pv = jnp.zeros((16,), jnp.int32)
            for i in range(16):
                b = _gather16(ev[k], jnp.full((16,), i, jnp.int32))
                cur = jnp.zeros((16,), jnp.int32)
                for j in range(nbin):
                    bj = b - 16 * j
                    inr = (jnp.where(bj >= 0, 1, 0)
                           * jnp.where(bj < 16, 1, 0))
                    g = _gather16(base[j], jnp.clip(bj, 0, 15))
                    cur = cur + inr * (g - cur)
                    base[j] = base[j] + inr * jnp.where(lane == bj, 1, 0)
                pv = jnp.where(lane == i, cur, pv)
            rank_vecs.append(pv)

        for j in range(nbin):
            counts_v[pl.ds(16 * j, 16)] = base[j]

        pltpu.sync_copy(counts_v, cnt_hbm.at[wid])
        plsc.subcore_barrier()
        pltpu.sync_copy(cnt_hbm, all_counts_v)

        tot = [jnp.zeros((16,), jnp.int32) for _ in range(nbin)]
        pre = [jnp.zeros((16,), jnp.int32) for _ in range(nbin)]
        for w in range(nw):
            for j in range(nbin):
                v = all_counts_v[w, pl.ds(16 * j, 16)]
                tot[j] = tot[j] + v
                pre[j] = pre[j] + jnp.where(w < wid, v, 0)

        carry = jnp.zeros((16,), jnp.int32)
        excl = []
        for j in range(nbin):
            cs = tot[j]
            for k in (1, 2, 4, 8):
                sh = _gather16(cs, jnp.maximum(lane - k, 0))
                cs = cs + jnp.where(lane >= k, sh, 0)
            excl.append(cs - tot[j] + carry)
            carry = carry + _gather16(cs, jnp.full((16,), 15, jnp.int32))

        @pl.when(wid == 0)
        def _():
            for j in range(nbin):
                soff_v[pl.ds(16 * j, 16)] = excl[j]
            soff_v[pl.ds(64, 16)] = jnp.full((16,), t, jnp.int32)
            for j in range(5, 8):
                soff_v[pl.ds(16 * j, 16)] = jnp.zeros((16,), jnp.int32)
            pltpu.sync_copy(soff_v, soff_hbm)

        base0 = [excl[j] + pre[j] for j in range(nbin)]
        for k in range(ch // 16):
            bj0 = ev[k]
            off = jnp.zeros((16,), jnp.int32)
            for j in range(nbin):
                bj = bj0 - 16 * j
                inr = (jnp.where(bj >= 0, 1, 0)
                       * jnp.where(bj < 16, 1, 0))
                g = _gather16(base0[j], jnp.clip(bj, 0, 15))
                off = off + inr * g
            pos_v[pl.ds(16 * k, 16)] = rank_vecs[k] + off

        pltpu.sync_copy(pos_v, pos_hbm.at[pl.ds(wid * ch, ch)])

    pos, soff, _ = dispatch(eid)
    return pos, soff


def _launder_body(x_ref, o_ref):
    o_ref[...] = x_ref[...]


def _launder(soff128):
    return pl.pallas_call(
        _launder_body,
        out_shape=jax.ShapeDtypeStruct((128,), jnp.int32),
    )(soff128)


def _schedule(soff, t):
    ntiles = t // TM
    nsteps = ntiles + E - 1

    starts = soff[:E]
    ends = soff[1:E + 1]
    counts = ends - starts

    nonzero = counts > 0
    t_first = starts // TM
    t_last = jnp.where(nonzero, (ends - 1) // TM, 0)
    ntile_e = jnp.where(nonzero, t_last - t_first + 1, 0)
    cume = jnp.cumsum(ntile_e)
    cume_ex = cume - ntile_e
    total = cume[-1]

    sidx = jnp.arange(nsteps, dtype=jnp.int32)
    e_arr = jnp.clip(jnp.searchsorted(cume, sidx, side='right'),
                     0, E - 1).astype(jnp.int32)
    m_arr = (t_first[e_arr] + sidx - cume_ex[e_arr]).astype(jnp.int32)
    active = sidx < total

    e_last = jnp.max(jnp.where(nonzero, jnp.arange(E, dtype=jnp.int32), -1))
    m_arr = jnp.where(active, m_arr, ntiles - 1)
    e_arr = jnp.where(active, e_arr, e_last)

    first = jnp.concatenate(
        [jnp.ones((1,), jnp.bool_), m_arr[1:] != m_arr[:-1]])
    sfl = (active.astype(jnp.int32) + 2 * first.astype(jnp.int32))
    sstart = starts[e_arr]
    send = ends[e_arr]
    return m_arr, e_arr, sfl, sstart, send


@jax.jit
def kernel(x, gate_w, W1, W3, W2, sw1, sw3, sw2):
    xf = x.reshape(-1, DIM)
    t = xf.shape[0]

    eid, g = _route(xf, gate_w, t)
    pos, soff128 = _dispatch(eid, t)
    soff = _launder(soff128)[:E + 1]
    sm, se, sfl, sstart, send = _schedule(soff, t)

    perm = jnp.zeros((t,), jnp.int32).at[pos].set(
        jnp.arange(t, dtype=jnp.int32), unique_indices=True)
    xs = jnp.take(xf, perm, axis=0)
    gb = g.astype(jnp.bfloat16).astype(jnp.float32)
    gs = jnp.take(gb, perm)
    g3 = gs.reshape(t // TM, 1, TM)

    out_sorted = _gmm(xs, W1, W3, W2, sw1, sw3, sw2, g3,
                      sm, se, sfl, sstart, send, t)
    out = out_sorted.at[pos].get(unique_indices=True,
                                 mode="promise_in_bounds")
    return out.reshape(x.shape)
